# baseline (speedup 1.0000x reference)
import functools

import jax
import jax.numpy as jnp
from jax import lax
from jax.experimental import pallas as pl
from jax.experimental.pallas import tpu as pltpu
from jax.experimental.pallas import tpu_sc as plsc

N = 10000
NP = 10240
E = 160000
D_IN = 256
H = 512
D_OUT = 256
EPS_LN = 1e-5

NSC = 2
NTILE = 16
CW = 64
RB = 1024
EB = 128
ROWS_PT = NP // NTILE

_MESH = plsc.VectorSubcoreMesh(core_axis_name="c", subcore_axis_name="s",
                               num_cores=NSC, num_subcores=NTILE)


def _zero_vmem(ref, rows, cols=128):
    z = jnp.zeros((16,), jnp.float32)

    def body(r, _):
        for q in range(cols // 16):
            ref[r, pl.ds(q * 16, 16)] = z
        return 0

    lax.fori_loop(0, rows, body, 0)


_DEG_TILES = 20
_DEG_EDGES = E // _DEG_TILES


def _deg_body(src_ref, dst_ref, out_ref, src_v, dst_v, acc_o, acc_i, zero_v,
              rid_o, rid_i, shared):
    c = lax.axis_index("c")
    s = lax.axis_index("s")
    w = c * NTILE + s

    _zero_vmem(zero_v, 16)
    _zero_vmem(acc_o, 80)
    _zero_vmem(acc_i, 80)

    base = lax.broadcasted_iota(jnp.int32, (16,), 0)
    for k in range(5):
        rid_o[pl.ds(k * 16, 16)] = base + (k * 16)
        rid_i[pl.ds(k * 16, 16)] = base + (80 + k * 16)

    @pl.when(s < 10)
    def _():
        pltpu.sync_copy(zero_v, shared.at[pl.ds(s * 16, 16)])

    plsc.subcore_barrier()

    @pl.when(w < _DEG_TILES)
    def _():
        eo = w * _DEG_EDGES
        pltpu.sync_copy(src_ref.at[pl.ds(eo, _DEG_EDGES)], src_v)
        pltpu.sync_copy(dst_ref.at[pl.ds(eo, _DEG_EDGES)], dst_v)
        ones = jnp.ones((16,), jnp.float32)

        def body(j, _):
            sv = src_v[pl.ds(j * 16, 16)]
            dv = dst_v[pl.ds(j * 16, 16)]
            plsc.addupdate_scatter(
                acc_o, [lax.shift_right_logical(sv, 7), sv & 127], ones)
            plsc.addupdate_scatter(
                acc_i, [lax.shift_right_logical(dv, 7), dv & 127], ones)
            return 0

        lax.fori_loop(0, _DEG_EDGES // 16, body, 0)
        pltpu.sync_copy(acc_o, shared.at[rid_o], add=True)
        pltpu.sync_copy(acc_i, shared.at[rid_i], add=True)

    plsc.subcore_barrier()

    @pl.when(s == 0)
    def _():
        pltpu.sync_copy(shared, out_ref.at[c])


_SC_PARAMS = pltpu.CompilerParams(needs_layout_passes=False,
                                  use_tc_tiling_on_sc=False)

_deg_call = pl.kernel(
    _deg_body,
    out_type=jax.ShapeDtypeStruct((NSC, 160, 128), jnp.float32),
    mesh=_MESH,
    compiler_params=_SC_PARAMS,
    scratch_types=[
        pltpu.VMEM((_DEG_EDGES,), jnp.int32),
        pltpu.VMEM((_DEG_EDGES,), jnp.int32),
        pltpu.VMEM((80, 128), jnp.float32),
        pltpu.VMEM((80, 128), jnp.float32),
        pltpu.VMEM((16, 128), jnp.float32),
        pltpu.VMEM((80,), jnp.int32),
        pltpu.VMEM((80,), jnp.int32),
        pltpu.VMEM_SHARED((160, 128), jnp.float32),
    ],
)


NEB = E // EB
NBM = NEB // NTILE
NBHI = NBM + 1
K = 3
NG = NBM // K
ZR = 32


def _conv_body(nchunk, tn_ref, src_ref, dst_ref, out_ref, idx_all, dst_all,
               rows, zero_v, shared, gsem, ssem, zsem):
    c = lax.axis_index("c")
    s = lax.axis_index("s")
    cps = nchunk // NSC
    r0 = s * NBM + jnp.maximum(s - (NTILE - 2), 0)
    tail = s >= NTILE - 2

    _zero_vmem(zero_v, ZR, CW)

    pltpu.sync_copy(src_ref.at[pl.ds(r0, NBHI)], idx_all)
    pltpu.sync_copy(dst_ref.at[pl.ds(r0, NBHI)], dst_all)

    def add_off(off):
        def rbody(r, _):
            for k in range(EB // 16):
                sl = pl.ds(k * 16, 16)
                idx_all[r, sl] = idx_all[r, sl] + off
            return 0
        lax.fori_loop(0, NBHI, rbody, 0)

    add_off(c * cps * NP)

    for local in range(cps):
        for zi in range(ROWS_PT // ZR):
            pltpu.async_copy(zero_v, shared.at[pl.ds(s * ROWS_PT + zi * ZR, ZR)],
                             zsem)
        for q in range(K):
            pltpu.async_copy(tn_ref.at[idx_all.at[q]], rows.at[q], gsem)
        for zi in range(ROWS_PT // ZR):
            pltpu.make_async_copy(zero_v, shared.at[pl.ds(0, ZR)], zsem).wait()
        plsc.subcore_barrier()

        def group(j, _):
            b0 = j * K

            def run(cur, oth):
                for q in range(K):
                    pltpu.make_async_copy(tn_ref.at[idx_all.at[0]],
                                          rows.at[cur + q], gsem).wait()
                    pltpu.async_copy(rows.at[cur + q],
                                     shared.at[dst_all.at[b0 + q]],
                                     ssem, add=True)

                @pl.when(b0 + K < NBM)
                def _():
                    @pl.when(j >= 1)
                    def _():
                        for q in range(K):
                            pltpu.make_async_copy(
                                rows.at[oth + q],
                                shared.at[pl.ds(0, EB)], ssem).wait()
                            pltpu.async_copy(tn_ref.at[idx_all.at[b0 + K + q]],
                                             rows.at[oth + q], gsem)

                    @pl.when(j == 0)
                    def _():
                        for q in range(K):
                            pltpu.async_copy(tn_ref.at[idx_all.at[b0 + K + q]],
                                             rows.at[oth + q], gsem)

            @pl.when(j % 2 == 0)
            def _():
                run(0, K)

            @pl.when(j % 2 == 1)
            def _():
                run(K, 0)

            return 0

        lax.fori_loop(0, NG, group, 0)
        for q in range(2 * K):
            pltpu.make_async_copy(rows.at[0], shared.at[pl.ds(0, EB)],
                                  ssem).wait()

        @pl.when(tail)
        def _():
            pltpu.async_copy(tn_ref.at[idx_all.at[NBM]], rows.at[0],
                             gsem).wait()
            pltpu.async_copy(rows.at[0], shared.at[dst_all.at[NBM]],
                             ssem, add=True).wait()

        if local + 1 < cps:
            add_off(NP)
        plsc.subcore_barrier()

        chunk = c * cps + local
        orow = pl.multiple_of(chunk * NP + s * ROWS_PT, 8)
        pltpu.sync_copy(shared.at[pl.ds(s * ROWS_PT, ROWS_PT)],
                        out_ref.at[pl.ds(orow, ROWS_PT)])


def _make_conv(nchunk):
    return pl.kernel(
        functools.partial(_conv_body, nchunk),
        out_type=jax.ShapeDtypeStruct((nchunk * NP, CW), jnp.float32),
        mesh=_MESH,
        compiler_params=_SC_PARAMS,
        scratch_types=[
            pltpu.VMEM((NBHI, EB), jnp.int32),
            pltpu.VMEM((NBHI, EB), jnp.int32),
            pltpu.VMEM((2 * K, EB, CW), jnp.float32),
            pltpu.VMEM((ZR, CW), jnp.float32),
            pltpu.VMEM_SHARED((NP, CW), jnp.float32),
            pltpu.SemaphoreType.DMA,
            pltpu.SemaphoreType.DMA,
            pltpu.SemaphoreType.DMA,
        ],
    )


_conv_h = _make_conv(H // CW)
_conv_o = _make_conv(D_OUT // CW)


def _tc_a_body(x_ref, wi_ref, bi_ref, wh_ref, dego_ref, tn_ref):
    h = jnp.dot(x_ref[...], wi_ref[...],
                preferred_element_type=jnp.float32) + bi_ref[...]
    t = jnp.dot(h, wh_ref[...], preferred_element_type=jnp.float32)
    sn = lax.rsqrt(jnp.maximum(dego_ref[0] + dego_ref[1], 1.0))
    tn = t * sn[:, None]
    for ci in range(H // CW):
        tn_ref[ci] = tn[:, ci * CW:(ci + 1) * CW]


def _tc_a(xp, W_in, b_in, W_h0, dego):
    grid = NP // RB
    return pl.pallas_call(
        _tc_a_body,
        grid=(grid,),
        in_specs=[
            pl.BlockSpec((RB, D_IN), lambda i: (i, 0)),
            pl.BlockSpec((D_IN, H), lambda i: (0, 0)),
            pl.BlockSpec((1, H), lambda i: (0, 0)),
            pl.BlockSpec((H, H), lambda i: (0, 0)),
            pl.BlockSpec((NSC, RB), lambda i: (0, i)),
        ],
        out_specs=pl.BlockSpec((H // CW, RB, CW), lambda i: (0, i, 0)),
        out_shape=jax.ShapeDtypeStruct((H // CW, NP, CW), jnp.float32),
    )(xp, W_in, b_in, W_h0, dego)


def _tc_b_body(nco, agg_ref, degi_ref, dego_ref, b_ref, g_ref, be_ref,
               w_ref, tn_ref):
    a = jnp.concatenate([agg_ref[ci] for ci in range(H // CW)], axis=1)
    dn = lax.rsqrt(jnp.maximum(degi_ref[0] + degi_ref[1], 1.0))
    hh = a * dn[:, None] + b_ref[...]
    mu = jnp.mean(hh, axis=1, keepdims=True)
    var = jnp.mean(jnp.square(hh - mu), axis=1, keepdims=True)
    hh = (hh - mu) * lax.rsqrt(var + EPS_LN) * g_ref[...] + be_ref[...]
    t = jnp.dot(hh, w_ref[...], preferred_element_type=jnp.float32)
    sn = lax.rsqrt(jnp.maximum(dego_ref[0] + dego_ref[1], 1.0))
    tn = t * sn[:, None]
    for ci in range(nco):
        tn_ref[ci] = tn[:, ci * CW:(ci + 1) * CW]


def _tc_b(agg, degi, dego, b_prev, g, be, W_next):
    grid = NP // RB
    d_next = W_next.shape[1]
    nco = d_next // CW
    return pl.pallas_call(
        functools.partial(_tc_b_body, nco),
        grid=(grid,),
        in_specs=[
            pl.BlockSpec((H // CW, RB, CW), lambda i: (0, i, 0)),
            pl.BlockSpec((NSC, RB), lambda i: (0, i)),
            pl.BlockSpec((NSC, RB), lambda i: (0, i)),
            pl.BlockSpec((1, H), lambda i: (0, 0)),
            pl.BlockSpec((1, H), lambda i: (0, 0)),
            pl.BlockSpec((1, H), lambda i: (0, 0)),
            pl.BlockSpec((H, d_next), lambda i: (0, 0)),
        ],
        out_specs=pl.BlockSpec((nco, RB, CW), lambda i: (0, i, 0)),
        out_shape=jax.ShapeDtypeStruct((nco, NP, CW), jnp.float32),
    )(agg, degi, dego, b_prev, g, be, W_next)


def _tc_c_body(agg_ref, degi_ref, bo_ref, out_ref):
    a = jnp.concatenate([agg_ref[ci] for ci in range(D_OUT // CW)], axis=1)
    dn = lax.rsqrt(jnp.maximum(degi_ref[0] + degi_ref[1], 1.0))
    v = a * dn[:, None] + bo_ref[...]
    nrm = jnp.sqrt(jnp.sum(v * v, axis=1, keepdims=True))
    out_ref[...] = v / jnp.maximum(nrm, 1e-12)


def _tc_c(agg, degi, b_out):
    grid = NP // RB
    return pl.pallas_call(
        _tc_c_body,
        grid=(grid,),
        in_specs=[
            pl.BlockSpec((D_OUT // CW, RB, CW), lambda i: (0, i, 0)),
            pl.BlockSpec((NSC, RB), lambda i: (0, i)),
            pl.BlockSpec((1, D_OUT), lambda i: (0, 0)),
        ],
        out_specs=pl.BlockSpec((RB, D_OUT), lambda i: (i, 0)),
        out_shape=jax.ShapeDtypeStruct((N, D_OUT), jnp.float32),
    )(agg, degi, b_out)


def kernel(x, edge_index, W_in, b_in, W_h0, b_h0, g0, be0, W_h1, b_h1,
           g1, be1, W_out, b_out):
    src = edge_index[0]
    dst = edge_index[1]
    src2 = src.reshape(NEB, EB)
    dst2 = dst.reshape(NEB, EB)

    degs = _deg_call(src, dst).reshape(NSC, 2, NP)
    dego = degs[:, 0, :]
    degi = degs[:, 1, :]

    nch = H // CW
    nco = D_OUT // CW
    tn0 = _tc_a(x, W_in, b_in.reshape(1, H), W_h0, dego)
    agg0 = _conv_h(tn0.reshape(nch * NP, CW), src2, dst2).reshape(nch, NP, CW)
    tn1 = _tc_b(agg0, degi, dego, b_h0.reshape(1, H), g0.reshape(1, H),
                be0.reshape(1, H), W_h1)
    agg1 = _conv_h(tn1.reshape(nch * NP, CW), src2, dst2).reshape(nch, NP, CW)
    tn2 = _tc_b(agg1, degi, dego, b_h1.reshape(1, H), g1.reshape(1, H),
                be1.reshape(1, H), W_out)
    agg2 = _conv_o(tn2.reshape(nco * NP, CW), src2, dst2).reshape(nco, NP, CW)
    return _tc_c(agg2, degi, b_out.reshape(1, D_OUT))


# speedup vs baseline: 1.0059x; 1.0059x over previous
import functools

import jax
import jax.numpy as jnp
from jax import lax
from jax.experimental import pallas as pl
from jax.experimental.pallas import tpu as pltpu
from jax.experimental.pallas import tpu_sc as plsc

N = 10000
NP = 10240
E = 160000
D_IN = 256
H = 512
D_OUT = 256
EPS_LN = 1e-5

NSC = 2
NTILE = 16
CW = 64
RB = 1024
EB = 80
EPT = E // NTILE
ROWS_PT = NP // NTILE

_MESH = plsc.VectorSubcoreMesh(core_axis_name="c", subcore_axis_name="s",
                               num_cores=NSC, num_subcores=NTILE)


def _zero_vmem(ref, rows, cols=128):
    z = jnp.zeros((16,), jnp.float32)

    def body(r, _):
        for q in range(cols // 16):
            ref[r, pl.ds(q * 16, 16)] = z
        return 0

    lax.fori_loop(0, rows, body, 0)


_DEG_TILES = 20
_DEG_EDGES = E // _DEG_TILES


def _deg_body(src_ref, dst_ref, out_ref, src_v, dst_v, acc_o, acc_i, zero_v,
              rid_o, rid_i, shared):
    c = lax.axis_index("c")
    s = lax.axis_index("s")
    w = c * NTILE + s

    _zero_vmem(zero_v, 16)
    _zero_vmem(acc_o, 80)
    _zero_vmem(acc_i, 80)

    base = lax.broadcasted_iota(jnp.int32, (16,), 0)
    for k in range(5):
        rid_o[pl.ds(k * 16, 16)] = base + (k * 16)
        rid_i[pl.ds(k * 16, 16)] = base + (80 + k * 16)

    @pl.when(s < 10)
    def _():
        pltpu.sync_copy(zero_v, shared.at[pl.ds(s * 16, 16)])

    plsc.subcore_barrier()

    @pl.when(w < _DEG_TILES)
    def _():
        eo = w * _DEG_EDGES
        pltpu.sync_copy(src_ref.at[pl.ds(eo, _DEG_EDGES)], src_v)
        pltpu.sync_copy(dst_ref.at[pl.ds(eo, _DEG_EDGES)], dst_v)
        ones = jnp.ones((16,), jnp.float32)

        def body(j, _):
            sv = src_v[pl.ds(j * 16, 16)]
            dv = dst_v[pl.ds(j * 16, 16)]
            plsc.addupdate_scatter(
                acc_o, [lax.shift_right_logical(sv, 7), sv & 127], ones)
            plsc.addupdate_scatter(
                acc_i, [lax.shift_right_logical(dv, 7), dv & 127], ones)
            return 0

        lax.fori_loop(0, _DEG_EDGES // 16, body, 0)
        pltpu.sync_copy(acc_o, shared.at[rid_o], add=True)
        pltpu.sync_copy(acc_i, shared.at[rid_i], add=True)

    plsc.subcore_barrier()

    @pl.when(s == 0)
    def _():
        pltpu.sync_copy(shared, out_ref.at[c])


_SC_PARAMS = pltpu.CompilerParams(needs_layout_passes=False,
                                  use_tc_tiling_on_sc=False)

_deg_call = pl.kernel(
    _deg_body,
    out_type=jax.ShapeDtypeStruct((NSC, 160, 128), jnp.float32),
    mesh=_MESH,
    compiler_params=_SC_PARAMS,
    scratch_types=[
        pltpu.VMEM((_DEG_EDGES,), jnp.int32),
        pltpu.VMEM((_DEG_EDGES,), jnp.int32),
        pltpu.VMEM((80, 128), jnp.float32),
        pltpu.VMEM((80, 128), jnp.float32),
        pltpu.VMEM((16, 128), jnp.float32),
        pltpu.VMEM((80,), jnp.int32),
        pltpu.VMEM((80,), jnp.int32),
        pltpu.VMEM_SHARED((160, 128), jnp.float32),
    ],
)


NB = EPT // EB
K = 5
NG = NB // K
ZR = 64


def _conv_body(nchunk, tn_ref, src_ref, dst_ref, out_ref, idx_all, dst_all,
               rows, zero_v, shared, gsem, ssem, zsem):
    c = lax.axis_index("c")
    s = lax.axis_index("s")
    cps = nchunk // NSC

    _zero_vmem(zero_v, ZR, CW)

    pltpu.sync_copy(src_ref.at[s], idx_all)
    pltpu.sync_copy(dst_ref.at[s], dst_all)

    def add_off(off):
        def rbody(r, _):
            for k in range(EB // 16):
                sl = pl.ds(k * 16, 16)
                idx_all[r, sl] = idx_all[r, sl] + off
            return 0
        lax.fori_loop(0, NB, rbody, 0)

    add_off(c * cps * NP)

    for local in range(cps):
        for zi in range(ROWS_PT // ZR):
            pltpu.async_copy(zero_v, shared.at[pl.ds(s * ROWS_PT + zi * ZR, ZR)],
                             zsem)
        for q in range(K):
            pltpu.async_copy(tn_ref.at[idx_all.at[q]], rows.at[q], gsem)
        for zi in range(ROWS_PT // ZR):
            pltpu.make_async_copy(zero_v, shared.at[pl.ds(0, ZR)], zsem).wait()
        plsc.subcore_barrier()

        def group(j, _):
            b0 = j * K

            def run(cur, oth):
                for q in range(K):
                    pltpu.make_async_copy(tn_ref.at[idx_all.at[0]],
                                          rows.at[cur + q], gsem).wait()
                    pltpu.async_copy(rows.at[cur + q],
                                     shared.at[dst_all.at[b0 + q]],
                                     ssem, add=True)

                @pl.when(b0 + K < NB)
                def _():
                    @pl.when(j >= 1)
                    def _():
                        for q in range(K):
                            pltpu.make_async_copy(
                                rows.at[oth + q],
                                shared.at[pl.ds(0, EB)], ssem).wait()
                            pltpu.async_copy(tn_ref.at[idx_all.at[b0 + K + q]],
                                             rows.at[oth + q], gsem)

                    @pl.when(j == 0)
                    def _():
                        for q in range(K):
                            pltpu.async_copy(tn_ref.at[idx_all.at[b0 + K + q]],
                                             rows.at[oth + q], gsem)

            @pl.when(j % 2 == 0)
            def _():
                run(0, K)

            @pl.when(j % 2 == 1)
            def _():
                run(K, 0)

            return 0

        lax.fori_loop(0, NG, group, 0)
        if local + 1 < cps:
            add_off(NP)
        for q in range(2 * K):
            pltpu.make_async_copy(rows.at[0], shared.at[pl.ds(0, EB)],
                                  ssem).wait()
        plsc.subcore_barrier()

        chunk = c * cps + local
        orow = pl.multiple_of(chunk * NP + s * ROWS_PT, 8)
        pltpu.sync_copy(shared.at[pl.ds(s * ROWS_PT, ROWS_PT)],
                        out_ref.at[pl.ds(orow, ROWS_PT)])


def _make_conv(nchunk):
    return pl.kernel(
        functools.partial(_conv_body, nchunk),
        out_type=jax.ShapeDtypeStruct((nchunk * NP, CW), jnp.float32),
        mesh=_MESH,
        compiler_params=_SC_PARAMS,
        scratch_types=[
            pltpu.VMEM((NB, EB), jnp.int32),
            pltpu.VMEM((NB, EB), jnp.int32),
            pltpu.VMEM((2 * K, EB, CW), jnp.float32),
            pltpu.VMEM((ZR, CW), jnp.float32),
            pltpu.VMEM_SHARED((NP, CW), jnp.float32),
            pltpu.SemaphoreType.DMA,
            pltpu.SemaphoreType.DMA,
            pltpu.SemaphoreType.DMA,
        ],
    )


_conv_h = _make_conv(H // CW)
_conv_o = _make_conv(D_OUT // CW)


def _tc_a_body(x_ref, wi_ref, bi_ref, wh_ref, dego_ref, tn_ref):
    h = jnp.dot(x_ref[...], wi_ref[...],
                preferred_element_type=jnp.float32) + bi_ref[...]
    t = jnp.dot(h, wh_ref[...], preferred_element_type=jnp.float32)
    sn = lax.rsqrt(jnp.maximum(dego_ref[0] + dego_ref[1], 1.0))
    tn = t * sn[:, None]
    for ci in range(H // CW):
        tn_ref[ci] = tn[:, ci * CW:(ci + 1) * CW]


def _tc_a(xp, W_in, b_in, W_h0, dego):
    grid = NP // RB
    return pl.pallas_call(
        _tc_a_body,
        grid=(grid,),
        in_specs=[
            pl.BlockSpec((RB, D_IN), lambda i: (i, 0)),
            pl.BlockSpec((D_IN, H), lambda i: (0, 0)),
            pl.BlockSpec((1, H), lambda i: (0, 0)),
            pl.BlockSpec((H, H), lambda i: (0, 0)),
            pl.BlockSpec((NSC, RB), lambda i: (0, i)),
        ],
        out_specs=pl.BlockSpec((H // CW, RB, CW), lambda i: (0, i, 0)),
        out_shape=jax.ShapeDtypeStruct((H // CW, NP, CW), jnp.float32),
    )(xp, W_in, b_in, W_h0, dego)


def _tc_b_body(nco, agg_ref, degi_ref, dego_ref, b_ref, g_ref, be_ref,
               w_ref, tn_ref):
    a = jnp.concatenate([agg_ref[ci] for ci in range(H // CW)], axis=1)
    dn = lax.rsqrt(jnp.maximum(degi_ref[0] + degi_ref[1], 1.0))
    hh = a * dn[:, None] + b_ref[...]
    mu = jnp.mean(hh, axis=1, keepdims=True)
    var = jnp.mean(jnp.square(hh - mu), axis=1, keepdims=True)
    hh = (hh - mu) * lax.rsqrt(var + EPS_LN) * g_ref[...] + be_ref[...]
    t = jnp.dot(hh, w_ref[...], preferred_element_type=jnp.float32)
    sn = lax.rsqrt(jnp.maximum(dego_ref[0] + dego_ref[1], 1.0))
    tn = t * sn[:, None]
    for ci in range(nco):
        tn_ref[ci] = tn[:, ci * CW:(ci + 1) * CW]


def _tc_b(agg, degi, dego, b_prev, g, be, W_next):
    grid = NP // RB
    d_next = W_next.shape[1]
    nco = d_next // CW
    return pl.pallas_call(
        functools.partial(_tc_b_body, nco),
        grid=(grid,),
        in_specs=[
            pl.BlockSpec((H // CW, RB, CW), lambda i: (0, i, 0)),
            pl.BlockSpec((NSC, RB), lambda i: (0, i)),
            pl.BlockSpec((NSC, RB), lambda i: (0, i)),
            pl.BlockSpec((1, H), lambda i: (0, 0)),
            pl.BlockSpec((1, H), lambda i: (0, 0)),
            pl.BlockSpec((1, H), lambda i: (0, 0)),
            pl.BlockSpec((H, d_next), lambda i: (0, 0)),
        ],
        out_specs=pl.BlockSpec((nco, RB, CW), lambda i: (0, i, 0)),
        out_shape=jax.ShapeDtypeStruct((nco, NP, CW), jnp.float32),
    )(agg, degi, dego, b_prev, g, be, W_next)


def _tc_c_body(agg_ref, degi_ref, bo_ref, out_ref):
    a = jnp.concatenate([agg_ref[ci] for ci in range(D_OUT // CW)], axis=1)
    dn = lax.rsqrt(jnp.maximum(degi_ref[0] + degi_ref[1], 1.0))
    v = a * dn[:, None] + bo_ref[...]
    nrm = jnp.sqrt(jnp.sum(v * v, axis=1, keepdims=True))
    out_ref[...] = v / jnp.maximum(nrm, 1e-12)


def _tc_c(agg, degi, b_out):
    grid = NP // RB
    return pl.pallas_call(
        _tc_c_body,
        grid=(grid,),
        in_specs=[
            pl.BlockSpec((D_OUT // CW, RB, CW), lambda i: (0, i, 0)),
            pl.BlockSpec((NSC, RB), lambda i: (0, i)),
            pl.BlockSpec((1, D_OUT), lambda i: (0, 0)),
        ],
        out_specs=pl.BlockSpec((RB, D_OUT), lambda i: (i, 0)),
        out_shape=jax.ShapeDtypeStruct((N, D_OUT), jnp.float32),
    )(agg, degi, b_out)


def kernel(x, edge_index, W_in, b_in, W_h0, b_h0, g0, be0, W_h1, b_h1,
           g1, be1, W_out, b_out):
    src = edge_index[0]
    dst = edge_index[1]
    src2 = src.reshape(NTILE, NB, EB)
    dst2 = dst.reshape(NTILE, NB, EB)

    degs = _deg_call(src, dst).reshape(NSC, 2, NP)
    dego = degs[:, 0, :]
    degi = degs[:, 1, :]

    nch = H // CW
    nco = D_OUT // CW
    tn0 = _tc_a(x, W_in, b_in.reshape(1, H), W_h0, dego)
    agg0 = _conv_h(tn0.reshape(nch * NP, CW), src2, dst2).reshape(nch, NP, CW)
    tn1 = _tc_b(agg0, degi, dego, b_h0.reshape(1, H), g0.reshape(1, H),
                be0.reshape(1, H), W_h1)
    agg1 = _conv_h(tn1.reshape(nch * NP, CW), src2, dst2).reshape(nch, NP, CW)
    tn2 = _tc_b(agg1, degi, dego, b_h1.reshape(1, H), g1.reshape(1, H),
                be1.reshape(1, H), W_out)
    agg2 = _conv_o(tn2.reshape(nco * NP, CW), src2, dst2).reshape(nco, NP, CW)
    return _tc_c(agg2, degi, b_out.reshape(1, D_OUT))
